# Initial kernel scaffold; baseline (speedup 1.0000x reference)
import jax
import jax.numpy as jnp
from jax.experimental import pallas as pl


def kernel(img, theta):
    raise NotImplementedError("write your pallas kernel here")


# speedup vs baseline: 1.5207x; 1.5207x over previous
import functools
import numpy as np

import jax
import jax.numpy as jnp
from jax.experimental import pallas as pl


def _roll(a, shift, axis):
    if shift == 0:
        return a
    n = a.shape[axis]
    s = shift % n
    lo = jax.lax.slice_in_dim(a, n - s, n, axis=axis)
    hi = jax.lax.slice_in_dim(a, 0, n - s, axis=axis)
    return jax.lax.concatenate([lo, hi], dimension=axis)


def _nms_kernel(img_ref, theta_ref, out_ref):
    g = img_ref[0, 0]
    th = theta_ref[0, 0]

    t = th * (180.0 / np.pi)
    t = jnp.where(t < 0, t + 180.0, t)
    k = jnp.round(t * (1.0 / 45.0))

    c0 = (k == 0.0) | (k == 4.0)
    c45 = k == 1.0
    c90 = k == 2.0

    gu = _roll(g, -1, 0)
    gd = _roll(g, 1, 0)

    n1 = jnp.where(
        c0, _roll(g, -1, 1),
        jnp.where(c45, _roll(gu, -1, 1), jnp.where(c90, gu, _roll(gu, 1, 1))),
    )
    n2 = jnp.where(
        c0, _roll(g, 1, 1),
        jnp.where(c45, _roll(gd, 1, 1), jnp.where(c90, gd, _roll(gd, -1, 1))),
    )

    H, W = g.shape
    xi = jax.lax.broadcasted_iota(jnp.int32, (H, W), 0)
    yi = jax.lax.broadcasted_iota(jnp.int32, (H, W), 1)
    interior = (xi >= 1) & (xi <= H - 2) & (yi >= 1) & (yi <= W - 2)

    keep = (g >= n1) & (g >= n2) & interior
    out_ref[0, 0] = jnp.where(keep, g, 0.0)


@jax.jit
def kernel(img, theta):
    return pl.pallas_call(
        _nms_kernel,
        out_shape=jax.ShapeDtypeStruct(img.shape, img.dtype),
    )(img, theta)
